# baseline (speedup 1.0000x reference)
import numpy as np
import jax
import jax.numpy as jnp
from jax.experimental import pallas as pl
from jax.experimental.pallas import tpu as pltpu

_VMEM_LIMIT = 64 * 1024 * 1024


def _build_w1_selector():
    U = np.zeros((4, 4, 2, 4, 4, 2, 4, 8, 8, 4), np.float32)
    for pq in range(4):
        p, q = pq >> 1, pq & 1
        for AB in range(4):
            A, Bs = AB >> 1, AB & 1
            for u in range(2):
                for v in range(2):
                    a = 2 * A + u - p
                    b = 2 * Bs + v - q
                    if 0 <= a < 2 and 0 <= b < 2:
                        for c in range(4):
                            for hh in range(4):
                                for ww in range(4):
                                    U[pq, AB, u, c, hh, v, ww,
                                      4 * a + hh, 4 * b + ww, c] = 1.0
    return U.reshape(4, 4 * 256, 256)


_W1_SEL = _build_w1_selector()
_S1 = (0, 1, 10, 11)
_S2 = (0, 1, 2, 10, 11, 12, 20, 21, 22)


def _shift_rows(t, s):
    if s == 0:
        return t
    pad = jnp.zeros((s, t.shape[1]), t.dtype)
    return jnp.concatenate([t[s:, :], pad], axis=0)


def _prep_kernel(w1_ref, w2_ref, u1_ref, o1_ref, o2_ref):
    planes = [jnp.dot(u1_ref[1024 * pq:1024 * (pq + 1), :], w1_ref[...],
                      preferred_element_type=jnp.float32)
              for pq in range(4)]
    o1_ref[...] = jnp.concatenate(planes, axis=1)
    zero64 = jnp.zeros((64, 32), jnp.float32)
    cols = []
    for k in range(9):
        ii, jj = k // 3, k % 3
        rows = []
        for pq in range(4):
            p, q = pq >> 1, pq & 1
            i, j = 2 * ii + p, 2 * jj + q
            if i < 5 and j < 5:
                r = (i * 5 + j) * 64
                rows.append(w2_ref[r:r + 64, :])
            else:
                rows.append(zero64)
        cols.append(jnp.concatenate(rows, axis=0))
    o2_ref[...] = jnp.concatenate(cols, axis=1)


def _conv_fused_kernel(x_ref, w1_ref, w2_ref, b1_ref, b2_ref, o_ref):
    x = x_ref[...]
    acc = None
    for k, s in enumerate(_S1):
        t = jnp.dot(x, w1_ref[256 * k:256 * (k + 1), :],
                    preferred_element_type=jnp.float32)
        t = _shift_rows(t, s)
        acc = t if acc is None else acc + t
    b1 = b1_ref[...]
    b1cat = jnp.concatenate([b1, b1, b1, b1], axis=1)
    y = jnp.maximum(acc + b1cat, 0.0)
    t2 = jnp.dot(y, w2_ref[...], preferred_element_type=jnp.float32)
    acc2 = None
    for k, s in enumerate(_S2):
        piece = _shift_rows(t2[:, 32 * k:32 * (k + 1)], s)
        acc2 = piece if acc2 is None else acc2 + piece
    o_ref[...] = jnp.maximum(acc2 + b2_ref[...], 0.0)


def _duel_head_fused_kernel(f_ref, wc_ref, bc_ref, wb_ref, bb_ref, o_ref):
    zpad = jnp.zeros((64, 128), jnp.float32)
    blocks = []
    for oh in range(8):
        blocks.append(wc_ref[256 * oh:256 * (oh + 1), :])
        blocks.append(zpad)
    blocks.append(jnp.zeros((640, 128), jnp.float32))
    wc_pad = jnp.concatenate(blocks, axis=0)
    h = jnp.maximum(
        jnp.dot(f_ref[...], wc_pad, preferred_element_type=jnp.float32)
        + bc_ref[...], 0.0)
    o2 = jnp.dot(h, wb_ref[...], preferred_element_type=jnp.float32) + bb_ref[...]
    v = o2[:, :1]
    rows, no = o2.shape
    a_mean = (jnp.sum(o2) - jnp.sum(v)) * (1.0 / (rows * (no - 1)))
    o_ref[...] = o2[:, 1:] + (v - a_mean)


def kernel(x_nchw, w1m, b1, w2m, b2, wcat, bcat, wblk, bblk):
    B = x_nchw.shape[0]
    bb = 16 if B % 16 == 0 else (8 if B % 8 == 0 else B)

    xs2 = (x_nchw.reshape(B, 4, 10, 2, 4, 10, 2, 4)
           .transpose(0, 2, 5, 3, 1, 4, 6, 7)
           .reshape(B * 100, 256))

    w1cat, w2cat = pl.pallas_call(
        _prep_kernel,
        out_shape=(jax.ShapeDtypeStruct((1024, 256), jnp.float32),
                   jax.ShapeDtypeStruct((256, 288), jnp.float32)),
        grid=(1,),
        in_specs=[
            pl.BlockSpec((256, 64), lambda i: (0, 0)),
            pl.BlockSpec((1600, 32), lambda i: (0, 0)),
            pl.BlockSpec((4096, 256), lambda i: (0, 0)),
        ],
        out_specs=(pl.BlockSpec((1024, 256), lambda i: (0, 0)),
                   pl.BlockSpec((256, 288), lambda i: (0, 0))),
        compiler_params=pltpu.CompilerParams(
            dimension_semantics=("arbitrary",),
            vmem_limit_bytes=_VMEM_LIMIT),
    )(w1m, w2m, jnp.asarray(_W1_SEL).reshape(4096, 256))

    z = pl.pallas_call(
        _conv_fused_kernel,
        out_shape=jax.ShapeDtypeStruct((B * 100, 32), jnp.float32),
        grid=(B // bb,),
        in_specs=[
            pl.BlockSpec((bb * 100, 256), lambda i: (i, 0)),
            pl.BlockSpec((1024, 256), lambda i: (0, 0)),
            pl.BlockSpec((256, 288), lambda i: (0, 0)),
            pl.BlockSpec((1, 64), lambda i: (0, 0)),
            pl.BlockSpec((1, 32), lambda i: (0, 0)),
        ],
        out_specs=pl.BlockSpec((bb * 100, 32), lambda i: (i, 0)),
        compiler_params=pltpu.CompilerParams(
            dimension_semantics=("parallel",),
            vmem_limit_bytes=_VMEM_LIMIT),
    )(xs2, w1cat, w2cat, b1.reshape(1, 64), b2.reshape(1, 32))

    feat = z.reshape(B, 3200)

    out = pl.pallas_call(
        _duel_head_fused_kernel,
        out_shape=jax.ShapeDtypeStruct((B, 6), jnp.float32),
        grid=(1,),
        in_specs=[
            pl.BlockSpec((B, 3200), lambda i: (0, 0)),
            pl.BlockSpec((2048, 128), lambda i: (0, 0)),
            pl.BlockSpec((1, 128), lambda i: (0, 0)),
            pl.BlockSpec((128, 7), lambda i: (0, 0)),
            pl.BlockSpec((1, 7), lambda i: (0, 0)),
        ],
        out_specs=pl.BlockSpec((B, 6), lambda i: (0, 0)),
        compiler_params=pltpu.CompilerParams(
            dimension_semantics=("arbitrary",),
            vmem_limit_bytes=_VMEM_LIMIT),
    )(feat, wcat, bcat.reshape(1, 128), wblk, bblk.reshape(1, 7))
    return out
